# Initial kernel scaffold; baseline (speedup 1.0000x reference)
import jax
import jax.numpy as jnp
from jax.experimental import pallas as pl


def kernel(x, edge_index, W1, b1, W2, b2, Wc, bc):
    raise NotImplementedError("write your pallas kernel here")


# speedup vs baseline: 7.6117x; 7.6117x over previous
import functools

import jax
import jax.numpy as jnp
from jax import lax
from jax.experimental import pallas as pl
from jax.experimental.pallas import tpu as pltpu
from jax.experimental.pallas import tpu_sc as plsc

N = 10000
E = 320000
D = 128
DOUT = 64

NC = 2
NS = 16
CHUNK = 128
NPAD = 10240
ROWS_PER_TILE = NPAD // NS
E_PAD = 327680
EDGES_PER_TILE = E_PAD // (NC * NS)
NCHUNK = EDGES_PER_TILE // CHUNK
ROW_BLK = 1000


def _sc_mesh():
    return plsc.VectorSubcoreMesh(core_axis_name="c", subcore_axis_name="s")


def _deg_pass(dstp, ones16, z16):
    k = functools.partial(
        pl.kernel,
        out_type=(
            jax.ShapeDtypeStruct((NPAD, 16), jnp.float32),
            jax.ShapeDtypeStruct((NPAD, 16), jnp.float32),
        ),
        mesh=_sc_mesh(),
        scratch_types=[
            pltpu.VMEM_SHARED((NPAD, 16), jnp.float32),
            pltpu.VMEM((CHUNK,), jnp.int32),
            pltpu.VMEM((CHUNK, 16), jnp.float32),
        ],
    )(_deg_body)
    return k(dstp, ones16, z16)


def _deg_body(dst_hbm, ones_hbm, z16_hbm, out_a, out_b, acc, dst_v, ones_v):
    cid = lax.axis_index("c")
    sid = lax.axis_index("s")
    wid = cid * NS + sid
    row0 = sid * ROWS_PER_TILE
    pltpu.sync_copy(z16_hbm, acc.at[pl.ds(row0, ROWS_PER_TILE), :])
    pltpu.sync_copy(ones_hbm, ones_v)
    plsc.subcore_barrier()

    def body(i, carry):
        base = pl.multiple_of(wid * EDGES_PER_TILE + i * CHUNK, CHUNK)
        pltpu.sync_copy(dst_hbm.at[pl.ds(base, CHUNK)], dst_v)
        pltpu.sync_copy(ones_v, acc.at[dst_v], add=True)
        return carry

    lax.fori_loop(0, NCHUNK, body, 0)
    plsc.subcore_barrier()

    @pl.when(cid == 0)
    def _():
        pltpu.sync_copy(acc.at[pl.ds(row0, ROWS_PER_TILE), :],
                        out_a.at[pl.ds(row0, ROWS_PER_TILE), :])

    @pl.when(cid == 1)
    def _():
        pltpu.sync_copy(acc.at[pl.ds(row0, ROWS_PER_TILE), :],
                        out_b.at[pl.ds(row0, ROWS_PER_TILE), :])


def _spmm_pass(g_pad, srcp, dstp, z128):
    k = functools.partial(
        pl.kernel,
        out_type=(
            jax.ShapeDtypeStruct((NPAD, D), jnp.float32),
            jax.ShapeDtypeStruct((NPAD, D), jnp.float32),
        ),
        mesh=_sc_mesh(),
        scratch_types=[
            pltpu.VMEM_SHARED((NPAD, D), jnp.float32),
            pltpu.VMEM((CHUNK,), jnp.int32),
            pltpu.VMEM((CHUNK,), jnp.int32),
            pltpu.VMEM((CHUNK, D), jnp.float32),
            pltpu.SemaphoreType.DMA,
        ],
    )(_spmm_body)
    return k(g_pad, srcp, dstp, z128)


def _spmm_body(g_hbm, src_hbm, dst_hbm, z_hbm, out_a, out_b,
               acc, src_v, dst_v, rows_v, sem):
    cid = lax.axis_index("c")
    sid = lax.axis_index("s")
    wid = cid * NS + sid
    row0 = sid * ROWS_PER_TILE
    pltpu.sync_copy(z_hbm, acc.at[pl.ds(row0, ROWS_PER_TILE), :])
    plsc.subcore_barrier()

    def body(i, carry):
        base = pl.multiple_of(wid * EDGES_PER_TILE + i * CHUNK, CHUNK)
        pltpu.sync_copy(src_hbm.at[pl.ds(base, CHUNK)], src_v)
        pltpu.sync_copy(dst_hbm.at[pl.ds(base, CHUNK)], dst_v)
        pltpu.async_copy(g_hbm.at[src_v], rows_v, sem).wait()
        pltpu.sync_copy(rows_v, acc.at[dst_v], add=True)
        return carry

    lax.fori_loop(0, NCHUNK, body, 0)
    plsc.subcore_barrier()

    @pl.when(cid == 0)
    def _():
        pltpu.sync_copy(acc.at[pl.ds(row0, ROWS_PER_TILE), :],
                        out_a.at[pl.ds(row0, ROWS_PER_TILE), :])

    @pl.when(cid == 1)
    def _():
        pltpu.sync_copy(acc.at[pl.ds(row0, ROWS_PER_TILE), :],
                        out_b.at[pl.ds(row0, ROWS_PER_TILE), :])


def _dinv_block(deg_a, deg_b):
    deg = deg_a[:, 0:1] + deg_b[:, 0:1] + 1.0
    return lax.rsqrt(deg)


def _tc1_body(x_ref, w1_ref, da_ref, db_ref, out_ref):
    dinv = _dinv_block(da_ref[...], db_ref[...])
    h = jnp.dot(x_ref[...], w1_ref[...], preferred_element_type=jnp.float32)
    out_ref[...] = h * dinv


def _tc1(x, w1, deg_a, deg_b):
    grid = N // ROW_BLK
    return pl.pallas_call(
        _tc1_body,
        grid=(grid,),
        in_specs=[
            pl.BlockSpec((ROW_BLK, D), lambda i: (i, 0)),
            pl.BlockSpec((D, D), lambda i: (0, 0)),
            pl.BlockSpec((ROW_BLK, 16), lambda i: (i, 0)),
            pl.BlockSpec((ROW_BLK, 16), lambda i: (i, 0)),
        ],
        out_specs=pl.BlockSpec((ROW_BLK, D), lambda i: (i, 0)),
        out_shape=jax.ShapeDtypeStruct((N, D), jnp.float32),
    )(x, w1, deg_a, deg_b)


def _tc2_body(pa_ref, pb_ref, g_ref, da_ref, db_ref, b_ref, w_ref, out_ref):
    dinv = _dinv_block(da_ref[...], db_ref[...])
    z = (pa_ref[...] + pb_ref[...] + g_ref[...]) * dinv + b_ref[...]
    h = jnp.maximum(z, 0.0)
    out_ref[...] = jnp.dot(h, w_ref[...],
                           preferred_element_type=jnp.float32) * dinv


def _tc2(pa, pb, g, deg_a, deg_b, b1, w2):
    grid = N // ROW_BLK
    return pl.pallas_call(
        _tc2_body,
        grid=(grid,),
        in_specs=[
            pl.BlockSpec((ROW_BLK, D), lambda i: (i, 0)),
            pl.BlockSpec((ROW_BLK, D), lambda i: (i, 0)),
            pl.BlockSpec((ROW_BLK, D), lambda i: (i, 0)),
            pl.BlockSpec((ROW_BLK, 16), lambda i: (i, 0)),
            pl.BlockSpec((ROW_BLK, 16), lambda i: (i, 0)),
            pl.BlockSpec((1, D), lambda i: (0, 0)),
            pl.BlockSpec((D, D), lambda i: (0, 0)),
        ],
        out_specs=pl.BlockSpec((ROW_BLK, D), lambda i: (i, 0)),
        out_shape=jax.ShapeDtypeStruct((N, D), jnp.float32),
    )(pa, pb, g, deg_a, deg_b, b1, w2)


def _tc3_body(pa_ref, pb_ref, g_ref, da_ref, db_ref, b_ref, w_ref, bc_ref,
              out_ref):
    dinv = _dinv_block(da_ref[...], db_ref[...])
    z = (pa_ref[...] + pb_ref[...] + g_ref[...]) * dinv + b_ref[...]
    h = jnp.maximum(z, 0.0)
    out_ref[...] = jnp.dot(h, w_ref[...],
                           preferred_element_type=jnp.float32) + bc_ref[...]


def _tc3(pa, pb, g, deg_a, deg_b, b2, wc, bc):
    grid = N // ROW_BLK
    return pl.pallas_call(
        _tc3_body,
        grid=(grid,),
        in_specs=[
            pl.BlockSpec((ROW_BLK, D), lambda i: (i, 0)),
            pl.BlockSpec((ROW_BLK, D), lambda i: (i, 0)),
            pl.BlockSpec((ROW_BLK, D), lambda i: (i, 0)),
            pl.BlockSpec((ROW_BLK, 16), lambda i: (i, 0)),
            pl.BlockSpec((ROW_BLK, 16), lambda i: (i, 0)),
            pl.BlockSpec((1, D), lambda i: (0, 0)),
            pl.BlockSpec((D, DOUT), lambda i: (0, 0)),
            pl.BlockSpec((1, DOUT), lambda i: (0, 0)),
        ],
        out_specs=pl.BlockSpec((ROW_BLK, DOUT), lambda i: (i, 0)),
        out_shape=jax.ShapeDtypeStruct((N, DOUT), jnp.float32),
    )(pa, pb, g, deg_a, deg_b, b2, wc, bc)


def kernel(x, edge_index, W1, b1, W2, b2, Wc, bc):
    pad = jnp.full((E_PAD - E,), N, dtype=edge_index.dtype)
    srcp = jnp.concatenate([edge_index[0], pad])
    dstp = jnp.concatenate([edge_index[1], pad])

    ones16 = jnp.ones((CHUNK, 16), jnp.float32)
    z16 = jnp.zeros((ROWS_PER_TILE, 16), jnp.float32)
    z128 = jnp.zeros((ROWS_PER_TILE, D), jnp.float32)

    deg_a, deg_b = _deg_pass(dstp, ones16, z16)

    g1 = _tc1(x, W1, deg_a, deg_b)
    g1p = jnp.pad(g1, ((0, NPAD - N), (0, 0)))
    pa, pb = _spmm_pass(g1p, srcp, dstp, z128)

    g2 = _tc2(pa, pb, g1, deg_a, deg_b, b1.reshape(1, D), W2)
    g2p = jnp.pad(g2, ((0, NPAD - N), (0, 0)))
    qa, qb = _spmm_pass(g2p, srcp, dstp, z128)

    return _tc3(qa, qb, g2, deg_a, deg_b, b2.reshape(1, D), Wc,
                bc.reshape(1, DOUT))


# speedup vs baseline: 8.5439x; 1.1225x over previous
import functools

import jax
import jax.numpy as jnp
from jax import lax
from jax.experimental import pallas as pl
from jax.experimental.pallas import tpu as pltpu
from jax.experimental.pallas import tpu_sc as plsc

N = 10000
E = 320000
D = 128
DOUT = 64

NC = 2
NS = 16
CHUNK = 128
NPAD = 10240
ROWS_PER_TILE = NPAD // NS
E_PAD = 327680
EDGES_PER_TILE = E_PAD // (NC * NS)
NCHUNK = EDGES_PER_TILE // CHUNK
ROW_BLK = 1000
NBUF = 2


def _sc_mesh():
    return plsc.VectorSubcoreMesh(core_axis_name="c", subcore_axis_name="s",
                                  num_cores=NC, num_subcores=NS)


def _deg_pass(dstp, ones16, z16):
    k = functools.partial(
        pl.kernel,
        out_type=(
            jax.ShapeDtypeStruct((NPAD, 16), jnp.float32),
            jax.ShapeDtypeStruct((NPAD, 16), jnp.float32),
        ),
        mesh=_sc_mesh(),
        scratch_types=[
            pltpu.VMEM_SHARED((NPAD, 16), jnp.float32),
            pltpu.VMEM((CHUNK,), jnp.int32),
            pltpu.VMEM((CHUNK,), jnp.int32),
            pltpu.VMEM((CHUNK, 16), jnp.float32),
            pltpu.SemaphoreType.DMA,
            pltpu.SemaphoreType.DMA,
        ],
    )(_deg_body)
    return k(dstp, ones16, z16)


def _deg_body(dst_hbm, ones_hbm, z16_hbm, out_a, out_b, acc, d0, d1, ones_v,
              i0, i1):
    cid = lax.axis_index("c")
    sid = lax.axis_index("s")
    wid = cid * NS + sid
    row0 = sid * ROWS_PER_TILE
    ebase = wid * EDGES_PER_TILE
    pltpu.sync_copy(z16_hbm, acc.at[pl.ds(row0, ROWS_PER_TILE), :])
    pltpu.sync_copy(ones_hbm, ones_v)
    plsc.subcore_barrier()

    def body(i, carry):
        base = pl.multiple_of(ebase + i * CHUNK, CHUNK)
        pltpu.sync_copy(dst_hbm.at[pl.ds(base, CHUNK)], d0)
        pltpu.sync_copy(ones_v, acc.at[d0], add=True)
        return carry

    lax.fori_loop(0, NCHUNK, body, 0)
    plsc.subcore_barrier()

    @pl.when(cid == 0)
    def _():
        pltpu.sync_copy(acc.at[pl.ds(row0, ROWS_PER_TILE), :],
                        out_a.at[pl.ds(row0, ROWS_PER_TILE), :])

    @pl.when(cid == 1)
    def _():
        pltpu.sync_copy(acc.at[pl.ds(row0, ROWS_PER_TILE), :],
                        out_b.at[pl.ds(row0, ROWS_PER_TILE), :])


def _spmm_pass(g_pad, srcp, dstp, z128):
    k = functools.partial(
        pl.kernel,
        out_type=(
            jax.ShapeDtypeStruct((NPAD, D), jnp.float32),
            jax.ShapeDtypeStruct((NPAD, D), jnp.float32),
        ),
        mesh=_sc_mesh(),
        scratch_types=(
            [pltpu.VMEM_SHARED((NPAD, D), jnp.float32)]
            + [pltpu.VMEM((CHUNK,), jnp.int32) for _ in range(2 * NBUF)]
            + [pltpu.VMEM((CHUNK, D), jnp.float32) for _ in range(NBUF)]
            + [pltpu.SemaphoreType.DMA for _ in range(2 * NBUF)]
        ),
    )(_spmm_body)
    return k(g_pad, srcp, dstp, z128)


def _spmm_body(g_hbm, src_hbm, dst_hbm, z_hbm, out_a, out_b,
               acc, sv0, sv1, dv0, dv1, r0, r1, is0, is1, gs0, gs1):
    cid = lax.axis_index("c")
    sid = lax.axis_index("s")
    wid = cid * NS + sid
    row0 = sid * ROWS_PER_TILE
    sbuf = [sv0, sv1]
    dbuf = [dv0, dv1]
    rows = [r0, r1]
    isem = [is0, is1]
    gsem = [gs0, gs1]
    ebase = wid * EDGES_PER_TILE
    pltpu.sync_copy(z_hbm, acc.at[pl.ds(row0, ROWS_PER_TILE), :])
    plsc.subcore_barrier()

    def body(j, carry):
        b = j * NBUF
        handles = []
        for t in range(NBUF):
            base = pl.multiple_of(ebase + (b + t) * CHUNK, CHUNK)
            pltpu.sync_copy(src_hbm.at[pl.ds(base, CHUNK)], sbuf[t])
            pltpu.sync_copy(dst_hbm.at[pl.ds(base, CHUNK)], dbuf[t])
            handles.append(
                pltpu.async_copy(g_hbm.at[sbuf[t]], rows[t], gsem[t]))
        for t in range(NBUF):
            handles[t].wait()
            pltpu.sync_copy(rows[t], acc.at[dbuf[t]], add=True)
        return carry

    lax.fori_loop(0, NCHUNK // NBUF, body, 0)
    plsc.subcore_barrier()

    @pl.when(cid == 0)
    def _():
        pltpu.sync_copy(acc.at[pl.ds(row0, ROWS_PER_TILE), :],
                        out_a.at[pl.ds(row0, ROWS_PER_TILE), :])

    @pl.when(cid == 1)
    def _():
        pltpu.sync_copy(acc.at[pl.ds(row0, ROWS_PER_TILE), :],
                        out_b.at[pl.ds(row0, ROWS_PER_TILE), :])


def _dinv_block(deg_a, deg_b):
    deg = deg_a[:, 0:1] + deg_b[:, 0:1] + 1.0
    return lax.rsqrt(deg)


def _tc1_body(x_ref, w1_ref, da_ref, db_ref, out_ref):
    dinv = _dinv_block(da_ref[...], db_ref[...])
    h = jnp.dot(x_ref[...], w1_ref[...], preferred_element_type=jnp.float32)
    out_ref[...] = h * dinv


def _tc1(x, w1, deg_a, deg_b):
    grid = N // ROW_BLK
    return pl.pallas_call(
        _tc1_body,
        grid=(grid,),
        in_specs=[
            pl.BlockSpec((ROW_BLK, D), lambda i: (i, 0)),
            pl.BlockSpec((D, D), lambda i: (0, 0)),
            pl.BlockSpec((ROW_BLK, 16), lambda i: (i, 0)),
            pl.BlockSpec((ROW_BLK, 16), lambda i: (i, 0)),
        ],
        out_specs=pl.BlockSpec((ROW_BLK, D), lambda i: (i, 0)),
        out_shape=jax.ShapeDtypeStruct((N, D), jnp.float32),
    )(x, w1, deg_a, deg_b)


def _tc2_body(pa_ref, pb_ref, g_ref, da_ref, db_ref, b_ref, w_ref, out_ref):
    dinv = _dinv_block(da_ref[...], db_ref[...])
    z = (pa_ref[...] + pb_ref[...] + g_ref[...]) * dinv + b_ref[...]
    h = jnp.maximum(z, 0.0)
    out_ref[...] = jnp.dot(h, w_ref[...],
                           preferred_element_type=jnp.float32) * dinv


def _tc2(pa, pb, g, deg_a, deg_b, b1, w2):
    grid = N // ROW_BLK
    return pl.pallas_call(
        _tc2_body,
        grid=(grid,),
        in_specs=[
            pl.BlockSpec((ROW_BLK, D), lambda i: (i, 0)),
            pl.BlockSpec((ROW_BLK, D), lambda i: (i, 0)),
            pl.BlockSpec((ROW_BLK, D), lambda i: (i, 0)),
            pl.BlockSpec((ROW_BLK, 16), lambda i: (i, 0)),
            pl.BlockSpec((ROW_BLK, 16), lambda i: (i, 0)),
            pl.BlockSpec((1, D), lambda i: (0, 0)),
            pl.BlockSpec((D, D), lambda i: (0, 0)),
        ],
        out_specs=pl.BlockSpec((ROW_BLK, D), lambda i: (i, 0)),
        out_shape=jax.ShapeDtypeStruct((N, D), jnp.float32),
    )(pa, pb, g, deg_a, deg_b, b1, w2)


def _tc3_body(pa_ref, pb_ref, g_ref, da_ref, db_ref, b_ref, w_ref, bc_ref,
              out_ref):
    dinv = _dinv_block(da_ref[...], db_ref[...])
    z = (pa_ref[...] + pb_ref[...] + g_ref[...]) * dinv + b_ref[...]
    h = jnp.maximum(z, 0.0)
    out_ref[...] = jnp.dot(h, w_ref[...],
                           preferred_element_type=jnp.float32) + bc_ref[...]


def _tc3(pa, pb, g, deg_a, deg_b, b2, wc, bc):
    grid = N // ROW_BLK
    return pl.pallas_call(
        _tc3_body,
        grid=(grid,),
        in_specs=[
            pl.BlockSpec((ROW_BLK, D), lambda i: (i, 0)),
            pl.BlockSpec((ROW_BLK, D), lambda i: (i, 0)),
            pl.BlockSpec((ROW_BLK, D), lambda i: (i, 0)),
            pl.BlockSpec((ROW_BLK, 16), lambda i: (i, 0)),
            pl.BlockSpec((ROW_BLK, 16), lambda i: (i, 0)),
            pl.BlockSpec((1, D), lambda i: (0, 0)),
            pl.BlockSpec((D, DOUT), lambda i: (0, 0)),
            pl.BlockSpec((1, DOUT), lambda i: (0, 0)),
        ],
        out_specs=pl.BlockSpec((ROW_BLK, DOUT), lambda i: (i, 0)),
        out_shape=jax.ShapeDtypeStruct((N, DOUT), jnp.float32),
    )(pa, pb, g, deg_a, deg_b, b2, wc, bc)


def kernel(x, edge_index, W1, b1, W2, b2, Wc, bc):
    pad = jnp.full((E_PAD - E,), N, dtype=edge_index.dtype)
    srcp = jnp.concatenate([edge_index[0], pad])
    dstp = jnp.concatenate([edge_index[1], pad])

    ones16 = jnp.ones((CHUNK, 16), jnp.float32)
    z16 = jnp.zeros((ROWS_PER_TILE, 16), jnp.float32)
    z128 = jnp.zeros((ROWS_PER_TILE, D), jnp.float32)

    deg_a, deg_b = _deg_pass(dstp, ones16, z16)

    g1 = _tc1(x, W1, deg_a, deg_b)
    g1p = jnp.pad(g1, ((0, NPAD - N), (0, 0)))
    pa, pb = _spmm_pass(g1p, srcp, dstp, z128)

    g2 = _tc2(pa, pb, g1, deg_a, deg_b, b1.reshape(1, D), W2)
    g2p = jnp.pad(g2, ((0, NPAD - N), (0, 0)))
    qa, qb = _spmm_pass(g2p, srcp, dstp, z128)

    return _tc3(qa, qb, g2, deg_a, deg_b, b2.reshape(1, D), Wc,
                bc.reshape(1, DOUT))


# speedup vs baseline: 8.5553x; 1.0013x over previous
import functools

import jax
import jax.numpy as jnp
from jax import lax
from jax.experimental import pallas as pl
from jax.experimental.pallas import tpu as pltpu
from jax.experimental.pallas import tpu_sc as plsc

N = 10000
E = 320000
D = 128
DOUT = 64

NC = 2
NS = 16
CHUNK = 128
NPAD = 10240
ROWS_PER_TILE = NPAD // NS
E_PAD = 327680
EDGES_PER_TILE = E_PAD // (NC * NS)
NCHUNK = EDGES_PER_TILE // CHUNK
ROW_BLK = 1000
NBUF = 2


def _sc_mesh():
    return plsc.VectorSubcoreMesh(core_axis_name="c", subcore_axis_name="s",
                                  num_cores=NC, num_subcores=NS)


def _deg_pass(dstp, ones16, z16):
    k = functools.partial(
        pl.kernel,
        out_type=(
            jax.ShapeDtypeStruct((NPAD, 16), jnp.float32),
            jax.ShapeDtypeStruct((NPAD, 16), jnp.float32),
        ),
        mesh=_sc_mesh(),
        scratch_types=[
            pltpu.VMEM_SHARED((NPAD, 16), jnp.float32),
            pltpu.VMEM((CHUNK,), jnp.int32),
            pltpu.VMEM((CHUNK,), jnp.int32),
            pltpu.VMEM((CHUNK, 16), jnp.float32),
            pltpu.SemaphoreType.DMA,
            pltpu.SemaphoreType.DMA,
        ],
    )(_deg_body)
    return k(dstp, ones16, z16)


def _deg_body(dst_hbm, ones_hbm, z16_hbm, out_a, out_b, acc, d0, d1, ones_v,
              i0, i1):
    cid = lax.axis_index("c")
    sid = lax.axis_index("s")
    wid = cid * NS + sid
    row0 = sid * ROWS_PER_TILE
    ebase = wid * EDGES_PER_TILE
    pltpu.sync_copy(z16_hbm, acc.at[pl.ds(row0, ROWS_PER_TILE), :])
    pltpu.sync_copy(ones_hbm, ones_v)
    plsc.subcore_barrier()

    def body(i, carry):
        base = pl.multiple_of(ebase + i * CHUNK, CHUNK)
        pltpu.sync_copy(dst_hbm.at[pl.ds(base, CHUNK)], d0)
        pltpu.sync_copy(ones_v, acc.at[d0], add=True)
        return carry

    lax.fori_loop(0, NCHUNK, body, 0)
    plsc.subcore_barrier()

    @pl.when(cid == 0)
    def _():
        pltpu.sync_copy(acc.at[pl.ds(row0, ROWS_PER_TILE), :],
                        out_a.at[pl.ds(row0, ROWS_PER_TILE), :])

    @pl.when(cid == 1)
    def _():
        pltpu.sync_copy(acc.at[pl.ds(row0, ROWS_PER_TILE), :],
                        out_b.at[pl.ds(row0, ROWS_PER_TILE), :])


def _spmm_pass(g_pad, srcp, dstp, z128):
    k = functools.partial(
        pl.kernel,
        out_type=(
            jax.ShapeDtypeStruct((NPAD, D), jnp.float32),
            jax.ShapeDtypeStruct((NPAD, D), jnp.float32),
        ),
        mesh=_sc_mesh(),
        scratch_types=(
            [pltpu.VMEM_SHARED((NPAD, D), jnp.float32)]
            + [pltpu.VMEM((CHUNK,), jnp.int32) for _ in range(2 * NBUF)]
            + [pltpu.VMEM((CHUNK, D), jnp.float32) for _ in range(NBUF)]
            + [pltpu.SemaphoreType.DMA for _ in range(3 * NBUF)]
        ),
    )(_spmm_body)
    return k(g_pad, srcp, dstp, z128)


def _spmm_body(g_hbm, src_hbm, dst_hbm, z_hbm, out_a, out_b,
               acc, sv0, sv1, dv0, dv1, r0, r1, ss0, ss1, ds0, ds1, gs0, gs1):
    cid = lax.axis_index("c")
    sid = lax.axis_index("s")
    wid = cid * NS + sid
    row0 = sid * ROWS_PER_TILE
    sbuf = [sv0, sv1]
    dbuf = [dv0, dv1]
    rows = [r0, r1]
    ssem = [ss0, ss1]
    dsem = [ds0, ds1]
    gsem = [gs0, gs1]
    ebase = wid * EDGES_PER_TILE
    pltpu.sync_copy(z_hbm, acc.at[pl.ds(row0, ROWS_PER_TILE), :])
    plsc.subcore_barrier()

    def body(j, carry):
        b = j * NBUF
        sh, dh = [], []
        for t in range(NBUF):
            base = pl.multiple_of(ebase + (b + t) * CHUNK, CHUNK)
            sh.append(pltpu.async_copy(src_hbm.at[pl.ds(base, CHUNK)],
                                       sbuf[t], ssem[t]))
            dh.append(pltpu.async_copy(dst_hbm.at[pl.ds(base, CHUNK)],
                                       dbuf[t], dsem[t]))
        gh = []
        for t in range(NBUF):
            sh[t].wait()
            gh.append(pltpu.async_copy(g_hbm.at[sbuf[t]], rows[t], gsem[t]))
        for t in range(NBUF):
            dh[t].wait()
            gh[t].wait()
            pltpu.sync_copy(rows[t], acc.at[dbuf[t]], add=True)
        return carry

    lax.fori_loop(0, NCHUNK // NBUF, body, 0)
    plsc.subcore_barrier()

    @pl.when(cid == 0)
    def _():
        pltpu.sync_copy(acc.at[pl.ds(row0, ROWS_PER_TILE), :],
                        out_a.at[pl.ds(row0, ROWS_PER_TILE), :])

    @pl.when(cid == 1)
    def _():
        pltpu.sync_copy(acc.at[pl.ds(row0, ROWS_PER_TILE), :],
                        out_b.at[pl.ds(row0, ROWS_PER_TILE), :])


def _dinv_block(deg_a, deg_b):
    deg = deg_a[:, 0:1] + deg_b[:, 0:1] + 1.0
    return lax.rsqrt(deg)


def _tc1_body(x_ref, w1_ref, da_ref, db_ref, out_ref):
    dinv = _dinv_block(da_ref[...], db_ref[...])
    h = jnp.dot(x_ref[...], w1_ref[...], preferred_element_type=jnp.float32)
    out_ref[...] = h * dinv


def _tc1(x, w1, deg_a, deg_b):
    grid = N // ROW_BLK
    return pl.pallas_call(
        _tc1_body,
        grid=(grid,),
        in_specs=[
            pl.BlockSpec((ROW_BLK, D), lambda i: (i, 0)),
            pl.BlockSpec((D, D), lambda i: (0, 0)),
            pl.BlockSpec((ROW_BLK, 16), lambda i: (i, 0)),
            pl.BlockSpec((ROW_BLK, 16), lambda i: (i, 0)),
        ],
        out_specs=pl.BlockSpec((ROW_BLK, D), lambda i: (i, 0)),
        out_shape=jax.ShapeDtypeStruct((N, D), jnp.float32),
    )(x, w1, deg_a, deg_b)


def _tc2_body(pa_ref, pb_ref, g_ref, da_ref, db_ref, b_ref, w_ref, out_ref):
    dinv = _dinv_block(da_ref[...], db_ref[...])
    z = (pa_ref[...] + pb_ref[...] + g_ref[...]) * dinv + b_ref[...]
    h = jnp.maximum(z, 0.0)
    out_ref[...] = jnp.dot(h, w_ref[...],
                           preferred_element_type=jnp.float32) * dinv


def _tc2(pa, pb, g, deg_a, deg_b, b1, w2):
    grid = N // ROW_BLK
    return pl.pallas_call(
        _tc2_body,
        grid=(grid,),
        in_specs=[
            pl.BlockSpec((ROW_BLK, D), lambda i: (i, 0)),
            pl.BlockSpec((ROW_BLK, D), lambda i: (i, 0)),
            pl.BlockSpec((ROW_BLK, D), lambda i: (i, 0)),
            pl.BlockSpec((ROW_BLK, 16), lambda i: (i, 0)),
            pl.BlockSpec((ROW_BLK, 16), lambda i: (i, 0)),
            pl.BlockSpec((1, D), lambda i: (0, 0)),
            pl.BlockSpec((D, D), lambda i: (0, 0)),
        ],
        out_specs=pl.BlockSpec((ROW_BLK, D), lambda i: (i, 0)),
        out_shape=jax.ShapeDtypeStruct((N, D), jnp.float32),
    )(pa, pb, g, deg_a, deg_b, b1, w2)


def _tc3_body(pa_ref, pb_ref, g_ref, da_ref, db_ref, b_ref, w_ref, bc_ref,
              out_ref):
    dinv = _dinv_block(da_ref[...], db_ref[...])
    z = (pa_ref[...] + pb_ref[...] + g_ref[...]) * dinv + b_ref[...]
    h = jnp.maximum(z, 0.0)
    out_ref[...] = jnp.dot(h, w_ref[...],
                           preferred_element_type=jnp.float32) + bc_ref[...]


def _tc3(pa, pb, g, deg_a, deg_b, b2, wc, bc):
    grid = N // ROW_BLK
    return pl.pallas_call(
        _tc3_body,
        grid=(grid,),
        in_specs=[
            pl.BlockSpec((ROW_BLK, D), lambda i: (i, 0)),
            pl.BlockSpec((ROW_BLK, D), lambda i: (i, 0)),
            pl.BlockSpec((ROW_BLK, D), lambda i: (i, 0)),
            pl.BlockSpec((ROW_BLK, 16), lambda i: (i, 0)),
            pl.BlockSpec((ROW_BLK, 16), lambda i: (i, 0)),
            pl.BlockSpec((1, D), lambda i: (0, 0)),
            pl.BlockSpec((D, DOUT), lambda i: (0, 0)),
            pl.BlockSpec((1, DOUT), lambda i: (0, 0)),
        ],
        out_specs=pl.BlockSpec((ROW_BLK, DOUT), lambda i: (i, 0)),
        out_shape=jax.ShapeDtypeStruct((N, DOUT), jnp.float32),
    )(pa, pb, g, deg_a, deg_b, b2, wc, bc)


def kernel(x, edge_index, W1, b1, W2, b2, Wc, bc):
    pad = jnp.full((E_PAD - E,), N, dtype=edge_index.dtype)
    srcp = jnp.concatenate([edge_index[0], pad])
    dstp = jnp.concatenate([edge_index[1], pad])

    ones16 = jnp.ones((CHUNK, 16), jnp.float32)
    z16 = jnp.zeros((ROWS_PER_TILE, 16), jnp.float32)
    z128 = jnp.zeros((ROWS_PER_TILE, D), jnp.float32)

    deg_a, deg_b = _deg_pass(dstp, ones16, z16)

    g1 = _tc1(x, W1, deg_a, deg_b)
    g1p = jnp.pad(g1, ((0, NPAD - N), (0, 0)))
    pa, pb = _spmm_pass(g1p, srcp, dstp, z128)

    g2 = _tc2(pa, pb, g1, deg_a, deg_b, b1.reshape(1, D), W2)
    g2p = jnp.pad(g2, ((0, NPAD - N), (0, 0)))
    qa, qb = _spmm_pass(g2p, srcp, dstp, z128)

    return _tc3(qa, qb, g2, deg_a, deg_b, b2.reshape(1, D), Wc,
                bc.reshape(1, DOUT))


# speedup vs baseline: 8.5795x; 1.0028x over previous
import functools

import jax
import jax.numpy as jnp
from jax import lax
from jax.experimental import pallas as pl
from jax.experimental.pallas import tpu as pltpu
from jax.experimental.pallas import tpu_sc as plsc

N = 10000
E = 320000
D = 128
DOUT = 64

NC = 2
NS = 16
CHUNK = 128
NPAD = 10240
ROWS_PER_TILE = NPAD // NS
E_PAD = 327680
EDGES_PER_TILE = E_PAD // (NC * NS)
NCHUNK = EDGES_PER_TILE // CHUNK
ROW_BLK = 1000
NBUF = 2


def _sc_mesh():
    return plsc.VectorSubcoreMesh(core_axis_name="c", subcore_axis_name="s",
                                  num_cores=NC, num_subcores=NS)


def _deg_pass(dstp, ones16, z16):
    k = functools.partial(
        pl.kernel,
        out_type=(
            jax.ShapeDtypeStruct((NPAD, 16), jnp.float32),
            jax.ShapeDtypeStruct((NPAD, 16), jnp.float32),
        ),
        mesh=_sc_mesh(),
        scratch_types=[
            pltpu.VMEM_SHARED((NPAD, 16), jnp.float32),
            pltpu.VMEM((CHUNK,), jnp.int32),
            pltpu.VMEM((CHUNK,), jnp.int32),
            pltpu.VMEM((CHUNK, 16), jnp.float32),
            pltpu.SemaphoreType.DMA,
            pltpu.SemaphoreType.DMA,
        ],
    )(_deg_body)
    return k(dstp, ones16, z16)


def _deg_body(dst_hbm, ones_hbm, z16_hbm, out_a, out_b, acc, d0, d1, ones_v,
              i0, i1):
    cid = lax.axis_index("c")
    sid = lax.axis_index("s")
    wid = cid * NS + sid
    row0 = sid * ROWS_PER_TILE
    ebase = wid * EDGES_PER_TILE
    pltpu.sync_copy(z16_hbm, acc.at[pl.ds(row0, ROWS_PER_TILE), :])
    pltpu.sync_copy(ones_hbm, ones_v)
    plsc.subcore_barrier()

    def body(i, carry):
        base = pl.multiple_of(ebase + i * CHUNK, CHUNK)
        pltpu.sync_copy(dst_hbm.at[pl.ds(base, CHUNK)], d0)
        pltpu.sync_copy(ones_v, acc.at[d0], add=True)
        return carry

    lax.fori_loop(0, NCHUNK, body, 0)
    plsc.subcore_barrier()

    @pl.when(cid == 0)
    def _():
        pltpu.sync_copy(acc.at[pl.ds(row0, ROWS_PER_TILE), :],
                        out_a.at[pl.ds(row0, ROWS_PER_TILE), :])

    @pl.when(cid == 1)
    def _():
        pltpu.sync_copy(acc.at[pl.ds(row0, ROWS_PER_TILE), :],
                        out_b.at[pl.ds(row0, ROWS_PER_TILE), :])


def _spmm_pass(g_pad, srcp, dstp, z128):
    k = functools.partial(
        pl.kernel,
        out_type=(
            jax.ShapeDtypeStruct((NPAD, D), jnp.float32),
            jax.ShapeDtypeStruct((NPAD, D), jnp.float32),
        ),
        mesh=_sc_mesh(),
        scratch_types=(
            [pltpu.VMEM_SHARED((NPAD, D), jnp.float32)]
            + [pltpu.VMEM((CHUNK,), jnp.int32) for _ in range(2 * NBUF)]
            + [pltpu.VMEM((CHUNK, D), jnp.float32) for _ in range(NBUF)]
            + [pltpu.SemaphoreType.DMA for _ in range(4 * NBUF)]
        ),
    )(_spmm_body)
    return k(g_pad, srcp, dstp, z128)


def _spmm_body(g_hbm, src_hbm, dst_hbm, z_hbm, out_a, out_b,
               acc, sv0, sv1, dv0, dv1, r0, r1, ss0, ss1, ds0, ds1, gs0, gs1,
               cs0, cs1):
    cid = lax.axis_index("c")
    sid = lax.axis_index("s")
    wid = cid * NS + sid
    row0 = sid * ROWS_PER_TILE
    sbuf = [sv0, sv1]
    dbuf = [dv0, dv1]
    rows = [r0, r1]
    ssem = [ss0, ss1]
    dsem = [ds0, ds1]
    gsem = [gs0, gs1]
    csem = [cs0, cs1]
    ebase = wid * EDGES_PER_TILE
    pltpu.sync_copy(z_hbm, acc.at[pl.ds(row0, ROWS_PER_TILE), :])
    plsc.subcore_barrier()

    def body(j, carry):
        b = j * NBUF
        sh, dh = [], []
        for t in range(NBUF):
            base = pl.multiple_of(ebase + (b + t) * CHUNK, CHUNK)
            sh.append(pltpu.async_copy(src_hbm.at[pl.ds(base, CHUNK)],
                                       sbuf[t], ssem[t]))
            dh.append(pltpu.async_copy(dst_hbm.at[pl.ds(base, CHUNK)],
                                       dbuf[t], dsem[t]))
        gh = []
        for t in range(NBUF):
            sh[t].wait()
            gh.append(pltpu.async_copy(g_hbm.at[sbuf[t]], rows[t], gsem[t]))
        ch = []
        for t in range(NBUF):
            dh[t].wait()
            gh[t].wait()
            ch.append(pltpu.async_copy(rows[t], acc.at[dbuf[t]], csem[t],
                                       add=True))
        for t in range(NBUF):
            ch[t].wait()
        return carry

    lax.fori_loop(0, NCHUNK // NBUF, body, 0)
    plsc.subcore_barrier()

    @pl.when(cid == 0)
    def _():
        pltpu.sync_copy(acc.at[pl.ds(row0, ROWS_PER_TILE), :],
                        out_a.at[pl.ds(row0, ROWS_PER_TILE), :])

    @pl.when(cid == 1)
    def _():
        pltpu.sync_copy(acc.at[pl.ds(row0, ROWS_PER_TILE), :],
                        out_b.at[pl.ds(row0, ROWS_PER_TILE), :])


def _dinv_block(deg_a, deg_b):
    deg = deg_a[:, 0:1] + deg_b[:, 0:1] + 1.0
    return lax.rsqrt(deg)


def _tc1_body(x_ref, w1_ref, da_ref, db_ref, out_ref):
    dinv = _dinv_block(da_ref[...], db_ref[...])
    h = jnp.dot(x_ref[...], w1_ref[...], preferred_element_type=jnp.float32)
    out_ref[...] = h * dinv


def _tc1(x, w1, deg_a, deg_b):
    grid = N // ROW_BLK
    return pl.pallas_call(
        _tc1_body,
        grid=(grid,),
        in_specs=[
            pl.BlockSpec((ROW_BLK, D), lambda i: (i, 0)),
            pl.BlockSpec((D, D), lambda i: (0, 0)),
            pl.BlockSpec((ROW_BLK, 16), lambda i: (i, 0)),
            pl.BlockSpec((ROW_BLK, 16), lambda i: (i, 0)),
        ],
        out_specs=pl.BlockSpec((ROW_BLK, D), lambda i: (i, 0)),
        out_shape=jax.ShapeDtypeStruct((N, D), jnp.float32),
    )(x, w1, deg_a, deg_b)


def _tc2_body(pa_ref, pb_ref, g_ref, da_ref, db_ref, b_ref, w_ref, out_ref):
    dinv = _dinv_block(da_ref[...], db_ref[...])
    z = (pa_ref[...] + pb_ref[...] + g_ref[...]) * dinv + b_ref[...]
    h = jnp.maximum(z, 0.0)
    out_ref[...] = jnp.dot(h, w_ref[...],
                           preferred_element_type=jnp.float32) * dinv


def _tc2(pa, pb, g, deg_a, deg_b, b1, w2):
    grid = N // ROW_BLK
    return pl.pallas_call(
        _tc2_body,
        grid=(grid,),
        in_specs=[
            pl.BlockSpec((ROW_BLK, D), lambda i: (i, 0)),
            pl.BlockSpec((ROW_BLK, D), lambda i: (i, 0)),
            pl.BlockSpec((ROW_BLK, D), lambda i: (i, 0)),
            pl.BlockSpec((ROW_BLK, 16), lambda i: (i, 0)),
            pl.BlockSpec((ROW_BLK, 16), lambda i: (i, 0)),
            pl.BlockSpec((1, D), lambda i: (0, 0)),
            pl.BlockSpec((D, D), lambda i: (0, 0)),
        ],
        out_specs=pl.BlockSpec((ROW_BLK, D), lambda i: (i, 0)),
        out_shape=jax.ShapeDtypeStruct((N, D), jnp.float32),
    )(pa, pb, g, deg_a, deg_b, b1, w2)


def _tc3_body(pa_ref, pb_ref, g_ref, da_ref, db_ref, b_ref, w_ref, bc_ref,
              out_ref):
    dinv = _dinv_block(da_ref[...], db_ref[...])
    z = (pa_ref[...] + pb_ref[...] + g_ref[...]) * dinv + b_ref[...]
    h = jnp.maximum(z, 0.0)
    out_ref[...] = jnp.dot(h, w_ref[...],
                           preferred_element_type=jnp.float32) + bc_ref[...]


def _tc3(pa, pb, g, deg_a, deg_b, b2, wc, bc):
    grid = N // ROW_BLK
    return pl.pallas_call(
        _tc3_body,
        grid=(grid,),
        in_specs=[
            pl.BlockSpec((ROW_BLK, D), lambda i: (i, 0)),
            pl.BlockSpec((ROW_BLK, D), lambda i: (i, 0)),
            pl.BlockSpec((ROW_BLK, D), lambda i: (i, 0)),
            pl.BlockSpec((ROW_BLK, 16), lambda i: (i, 0)),
            pl.BlockSpec((ROW_BLK, 16), lambda i: (i, 0)),
            pl.BlockSpec((1, D), lambda i: (0, 0)),
            pl.BlockSpec((D, DOUT), lambda i: (0, 0)),
            pl.BlockSpec((1, DOUT), lambda i: (0, 0)),
        ],
        out_specs=pl.BlockSpec((ROW_BLK, DOUT), lambda i: (i, 0)),
        out_shape=jax.ShapeDtypeStruct((N, DOUT), jnp.float32),
    )(pa, pb, g, deg_a, deg_b, b2, wc, bc)


def kernel(x, edge_index, W1, b1, W2, b2, Wc, bc):
    pad = jnp.full((E_PAD - E,), N, dtype=edge_index.dtype)
    srcp = jnp.concatenate([edge_index[0], pad])
    dstp = jnp.concatenate([edge_index[1], pad])

    ones16 = jnp.ones((CHUNK, 16), jnp.float32)
    z16 = jnp.zeros((ROWS_PER_TILE, 16), jnp.float32)
    z128 = jnp.zeros((ROWS_PER_TILE, D), jnp.float32)

    deg_a, deg_b = _deg_pass(dstp, ones16, z16)

    g1 = _tc1(x, W1, deg_a, deg_b)
    g1p = jnp.pad(g1, ((0, NPAD - N), (0, 0)))
    pa, pb = _spmm_pass(g1p, srcp, dstp, z128)

    g2 = _tc2(pa, pb, g1, deg_a, deg_b, b1.reshape(1, D), W2)
    g2p = jnp.pad(g2, ((0, NPAD - N), (0, 0)))
    qa, qb = _spmm_pass(g2p, srcp, dstp, z128)

    return _tc3(qa, qb, g2, deg_a, deg_b, b2.reshape(1, D), Wc,
                bc.reshape(1, DOUT))
